# Initial kernel scaffold; baseline (speedup 1.0000x reference)
import jax
import jax.numpy as jnp
from jax.experimental import pallas as pl


def kernel(sess_embed, query, W1_w, W1_b, W2_w, W2_b, alpha_w, alpha_b, sections):
    raise NotImplementedError("write your pallas kernel here")


# speedup vs baseline: 7.6936x; 7.6936x over previous
import jax
import jax.numpy as jnp
from jax.experimental import pallas as pl


def _seq_query_block(e_ref, q_ref, w1t_ref, w2t_ref, b12_ref, aw_ref,
                     ab_ref, out_ref):
    e = e_ref[...]
    qw = jnp.dot(q_ref[0], w1t_ref[...],
                 preferred_element_type=jnp.float32) + b12_ref[...]
    h = jax.nn.sigmoid(
        jnp.dot(e, w2t_ref[...], preferred_element_type=jnp.float32) + qw)
    g = jax.lax.dot_general(h, e, (((0,), (0,)), ((), ())),
                            preferred_element_type=jnp.float32)
    esum = jnp.sum(e, axis=0, keepdims=True)
    out_ref[0] = (jnp.dot(aw_ref[...], g, preferred_element_type=jnp.float32)
                  + ab_ref[0, 0] * esum)


def kernel(sess_embed, query, W1_w, W1_b, W2_w, W2_b, alpha_w, alpha_b,
           sections):
    N, d = sess_embed.shape
    B = query.shape[0]
    S = N // B

    w1t = W1_w.T
    w2t = W2_w.T
    b12 = (W1_b + W2_b).reshape(1, d)
    ab = alpha_b.reshape(1, 1)

    out = pl.pallas_call(
        _seq_query_block,
        grid=(B,),
        in_specs=[
            pl.BlockSpec((S, d), lambda b: (b, 0)),
            pl.BlockSpec((1, 1, d), lambda b: (b, 0, 0)),
            pl.BlockSpec((d, d), lambda b: (0, 0)),
            pl.BlockSpec((d, d), lambda b: (0, 0)),
            pl.BlockSpec((1, d), lambda b: (0, 0)),
            pl.BlockSpec((1, d), lambda b: (0, 0)),
            pl.BlockSpec((1, 1), lambda b: (0, 0)),
        ],
        out_specs=pl.BlockSpec((1, 1, d), lambda b: (b, 0, 0)),
        out_shape=jax.ShapeDtypeStruct((B, 1, d), jnp.float32),
    )(sess_embed, query.reshape(B, 1, d), w1t, w2t, b12, alpha_w, ab)
    return out.reshape(B, d)


# speedup vs baseline: 7.7042x; 1.0014x over previous
import jax
import jax.numpy as jnp
from jax.experimental import pallas as pl
from jax.experimental.pallas import tpu as pltpu


def _seq_query_block(e_ref, q_ref, w1t_ref, w2t_ref, b12_ref, aw_ref,
                     ab_ref, out_ref):
    e = e_ref[...]
    qw = jnp.dot(q_ref[0], w1t_ref[...],
                 preferred_element_type=jnp.float32) + b12_ref[...]
    h = jax.nn.sigmoid(
        jnp.dot(e, w2t_ref[...], preferred_element_type=jnp.float32) + qw)
    g = jax.lax.dot_general(h, e, (((0,), (0,)), ((), ())),
                            preferred_element_type=jnp.float32)
    esum = jnp.sum(e, axis=0, keepdims=True)
    out_ref[0] = (jnp.dot(aw_ref[...], g, preferred_element_type=jnp.float32)
                  + ab_ref[0, 0] * esum)


def kernel(sess_embed, query, W1_w, W1_b, W2_w, W2_b, alpha_w, alpha_b,
           sections):
    N, d = sess_embed.shape
    B = query.shape[0]
    S = N // B

    w1t = W1_w.T
    w2t = W2_w.T
    b12 = (W1_b + W2_b).reshape(1, d)
    ab = alpha_b.reshape(1, 1)

    out = pl.pallas_call(
        _seq_query_block,
        grid=(B,),
        in_specs=[
            pl.BlockSpec((S, d), lambda b: (b, 0)),
            pl.BlockSpec((1, 1, d), lambda b: (b, 0, 0)),
            pl.BlockSpec((d, d), lambda b: (0, 0)),
            pl.BlockSpec((d, d), lambda b: (0, 0)),
            pl.BlockSpec((1, d), lambda b: (0, 0)),
            pl.BlockSpec((1, d), lambda b: (0, 0)),
            pl.BlockSpec((1, 1), lambda b: (0, 0)),
        ],
        out_specs=pl.BlockSpec((1, 1, d), lambda b: (b, 0, 0)),
        out_shape=jax.ShapeDtypeStruct((B, 1, d), jnp.float32),
        compiler_params=pltpu.CompilerParams(
            dimension_semantics=("parallel",)),
    )(sess_embed, query.reshape(B, 1, d), w1t, w2t, b12, alpha_w, ab)
    return out.reshape(B, d)


# speedup vs baseline: 8.6896x; 1.1279x over previous
import jax
import jax.numpy as jnp
from jax.experimental import pallas as pl
from jax.experimental.pallas import tpu as pltpu


def _seq_query_block(e_ref, q_ref, w1_ref, w2_ref, b1_ref, b2_ref, aw_ref,
                     ab_ref, out_ref):
    b = pl.program_id(0)
    e = e_ref[...]
    q = q_ref[pl.ds(b, 1), :]
    qw = jax.lax.dot_general(q, w1_ref[...], (((1,), (1,)), ((), ())),
                             preferred_element_type=jnp.float32)
    qw = qw + b1_ref[...] + b2_ref[...]
    z = jax.lax.dot_general(e, w2_ref[...], (((1,), (1,)), ((), ())),
                            preferred_element_type=jnp.float32)
    h = jax.nn.sigmoid(z + qw)
    g = jax.lax.dot_general(h, e, (((0,), (0,)), ((), ())),
                            preferred_element_type=jnp.float32)
    esum = jnp.sum(e, axis=0, keepdims=True)
    out_ref[pl.ds(b, 1), :] = (
        jnp.dot(aw_ref[...], g, preferred_element_type=jnp.float32)
        + ab_ref[0, 0] * esum)


def kernel(sess_embed, query, W1_w, W1_b, W2_w, W2_b, alpha_w, alpha_b,
           sections):
    N, d = sess_embed.shape
    B = query.shape[0]
    S = N // B

    return pl.pallas_call(
        _seq_query_block,
        grid=(B,),
        in_specs=[
            pl.BlockSpec((S, d), lambda b: (b, 0)),
            pl.BlockSpec((B, d), lambda b: (0, 0)),
            pl.BlockSpec((d, d), lambda b: (0, 0)),
            pl.BlockSpec((d, d), lambda b: (0, 0)),
            pl.BlockSpec((1, d), lambda b: (0, 0)),
            pl.BlockSpec((1, d), lambda b: (0, 0)),
            pl.BlockSpec((1, d), lambda b: (0, 0)),
            pl.BlockSpec((1, 1), lambda b: (0, 0)),
        ],
        out_specs=pl.BlockSpec((B, d), lambda b: (0, 0)),
        out_shape=jax.ShapeDtypeStruct((B, d), jnp.float32),
        compiler_params=pltpu.CompilerParams(
            dimension_semantics=("arbitrary",)),
    )(sess_embed, query, W1_w, W2_w, W1_b.reshape(1, d), W2_b.reshape(1, d),
      alpha_w, alpha_b.reshape(1, 1))


# speedup vs baseline: 11.1457x; 1.2826x over previous
import functools

import jax
import jax.numpy as jnp
from jax.experimental import pallas as pl
from jax.experimental.pallas import tpu as pltpu


def _seq_query_block(e_ref, q_ref, w1_ref, w2_ref, b1_ref, b2_ref, aw_ref,
                     ab_ref, out_ref, *, seg_per_block, seg_len):
    blk = pl.program_id(0)
    e = e_ref[...]
    q = q_ref[pl.ds(blk * seg_per_block, seg_per_block), :]
    qw = jax.lax.dot_general(q, w1_ref[...], (((1,), (1,)), ((), ())),
                             preferred_element_type=jnp.float32)
    qw = qw + b1_ref[...] + b2_ref[...]
    z = jax.lax.dot_general(e, w2_ref[...], (((1,), (1,)), ((), ())),
                            preferred_element_type=jnp.float32)
    rows = []
    for s in range(seg_per_block):
        lo = s * seg_len
        hs = jax.nn.sigmoid(z[lo:lo + seg_len] + qw[s:s + 1])
        es = e[lo:lo + seg_len]
        g = jax.lax.dot_general(hs, es, (((0,), (0,)), ((), ())),
                                preferred_element_type=jnp.float32)
        esum = jnp.sum(es, axis=0, keepdims=True)
        rows.append(
            jnp.dot(aw_ref[...], g, preferred_element_type=jnp.float32)
            + ab_ref[0, 0] * esum)
    out_ref[pl.ds(blk * seg_per_block, seg_per_block), :] = (
        jnp.concatenate(rows, axis=0))


def kernel(sess_embed, query, W1_w, W1_b, W2_w, W2_b, alpha_w, alpha_b,
           sections):
    N, d = sess_embed.shape
    B = query.shape[0]
    S = N // B
    SB = 4
    body = functools.partial(_seq_query_block, seg_per_block=SB, seg_len=S)

    return pl.pallas_call(
        body,
        grid=(B // SB,),
        in_specs=[
            pl.BlockSpec((SB * S, d), lambda b: (b, 0)),
            pl.BlockSpec((B, d), lambda b: (0, 0)),
            pl.BlockSpec((d, d), lambda b: (0, 0)),
            pl.BlockSpec((d, d), lambda b: (0, 0)),
            pl.BlockSpec((1, d), lambda b: (0, 0)),
            pl.BlockSpec((1, d), lambda b: (0, 0)),
            pl.BlockSpec((1, d), lambda b: (0, 0)),
            pl.BlockSpec((1, 1), lambda b: (0, 0)),
        ],
        out_specs=pl.BlockSpec((B, d), lambda b: (0, 0)),
        out_shape=jax.ShapeDtypeStruct((B, d), jnp.float32),
        compiler_params=pltpu.CompilerParams(
            dimension_semantics=("arbitrary",)),
    )(sess_embed, query, W1_w, W2_w, W1_b.reshape(1, d), W2_b.reshape(1, d),
      alpha_w, alpha_b.reshape(1, 1))
